# baseline (speedup 1.0000x reference)
import jax
import jax.numpy as jnp
from jax.experimental import pallas as pl
from jax.experimental.pallas import tpu as pltpu

_CHUNKS = 16


def _copy_body(ei_ref, ew_ref, ei_out, ew_out, sem_ei, sem_ew):
    re_ = ei_ref.shape[0] // _CHUNKS
    rw = ew_ref.shape[0] // _CHUNKS
    copies = []
    for k in range(_CHUNKS):
        ce = pltpu.make_async_copy(
            ei_ref.at[pl.ds(k * re_, re_), :],
            ei_out.at[pl.ds(k * re_, re_), :],
            sem_ei.at[k],
        )
        cw = pltpu.make_async_copy(
            ew_ref.at[pl.ds(k * rw, rw), :],
            ew_out.at[pl.ds(k * rw, rw), :],
            sem_ew.at[k],
        )
        ce.start()
        cw.start()
        copies.append((ce, cw))
    for ce, cw in copies:
        ce.wait()
        cw.wait()


def kernel(ei, ew):
    ei2 = ei.reshape(ei.size // 128, 128)
    ew2 = ew.reshape(ew.size // 128, 128)
    out = pl.pallas_call(
        _copy_body,
        in_specs=(
            pl.BlockSpec(memory_space=pl.ANY),
            pl.BlockSpec(memory_space=pl.ANY),
        ),
        out_specs=(
            pl.BlockSpec(memory_space=pl.ANY),
            pl.BlockSpec(memory_space=pl.ANY),
        ),
        out_shape=(
            jax.ShapeDtypeStruct(ei2.shape, ei2.dtype),
            jax.ShapeDtypeStruct(ew2.shape, ew2.dtype),
        ),
        scratch_shapes=(
            pltpu.SemaphoreType.DMA((_CHUNKS,)),
            pltpu.SemaphoreType.DMA((_CHUNKS,)),
        ),
    )(ei2, ew2)
    return out[0].reshape(ei.shape), out[1].reshape(ew.shape)


# speedup vs baseline: 10.7615x; 10.7615x over previous
import jax
import jax.numpy as jnp
from jax.experimental import pallas as pl
from jax.experimental.pallas import tpu as pltpu

_K = 25
_NBUF = 6
_DELAY = 2


def _copy_body(ei_ref, ew_ref, ei_out, ew_out,
               ei_buf, ew_buf, sei_in, sei_out, sew_in, sew_out):
    re_ = ei_ref.shape[0] // _K
    rw = ew_ref.shape[0] // _K

    def in_copies(k):
        s = k % _NBUF
        return (
            pltpu.make_async_copy(
                ei_ref.at[pl.ds(k * re_, re_), :], ei_buf.at[s], sei_in.at[s]),
            pltpu.make_async_copy(
                ew_ref.at[pl.ds(k * rw, rw), :], ew_buf.at[s], sew_in.at[s]),
        )

    def out_copies(k):
        s = k % _NBUF
        return (
            pltpu.make_async_copy(
                ei_buf.at[s], ei_out.at[pl.ds(k * re_, re_), :], sei_out.at[s]),
            pltpu.make_async_copy(
                ew_buf.at[s], ew_out.at[pl.ds(k * rw, rw), :], sew_out.at[s]),
        )

    for k in range(min(_NBUF, _K)):
        for c in in_copies(k):
            c.start()
    for k in range(_K):
        for c in in_copies(k):
            c.wait()
        for c in out_copies(k):
            c.start()
        j = k - _DELAY
        if j >= 0 and j + _NBUF < _K:
            for c in out_copies(j):
                c.wait()
            for c in in_copies(j + _NBUF):
                c.start()
    for j in range(max(0, _K - _NBUF), _K):
        for c in out_copies(j):
            c.wait()


def kernel(ei, ew):
    ei2 = ei.reshape(ei.size // 128, 128)
    ew2 = ew.reshape(ew.size // 128, 128)
    re_ = ei2.shape[0] // _K
    rw = ew2.shape[0] // _K
    out = pl.pallas_call(
        _copy_body,
        in_specs=(
            pl.BlockSpec(memory_space=pl.ANY),
            pl.BlockSpec(memory_space=pl.ANY),
        ),
        out_specs=(
            pl.BlockSpec(memory_space=pl.ANY),
            pl.BlockSpec(memory_space=pl.ANY),
        ),
        out_shape=(
            jax.ShapeDtypeStruct(ei2.shape, ei2.dtype),
            jax.ShapeDtypeStruct(ew2.shape, ew2.dtype),
        ),
        scratch_shapes=(
            pltpu.VMEM((_NBUF, re_, 128), ei.dtype),
            pltpu.VMEM((_NBUF, rw, 128), ew.dtype),
            pltpu.SemaphoreType.DMA((_NBUF,)),
            pltpu.SemaphoreType.DMA((_NBUF,)),
            pltpu.SemaphoreType.DMA((_NBUF,)),
            pltpu.SemaphoreType.DMA((_NBUF,)),
        ),
    )(ei2, ew2)
    return out[0].reshape(ei.shape), out[1].reshape(ew.shape)
